# baseline (speedup 1.0000x reference)
import jax
import jax.numpy as jnp
from jax.experimental import pallas as pl
from jax.experimental.pallas import tpu as pltpu

TWO_PI = 6.283185307179586
EPS = 1e-6
ROWS_PER_EVENT = 16


def _hawkes_body(x_ref, t_ref, xr_ref, tr_ref, px_ref, pt_ref, cov_ref,
                 z_ref, xg_ref, tg_ref, beta_ref, brep_ref, scal_ref,
                 log_ref, cross_ref, base_ref):
    alpha = scal_ref[0, 0]
    sigma = scal_ref[0, 1]
    omega = scal_ref[0, 2]
    inv2s2 = -0.5 / (sigma * sigma)
    snorm = 1.0 / (TWO_PI * sigma * sigma)

    pxb = px_ref[:, :]
    x0c = xr_ref[:, 0:1]
    x1c = xr_ref[:, 1:2]
    parity = jax.lax.broadcasted_iota(jnp.int32, pxb.shape, 1) % 2
    xrep = jnp.where(parity == 0, x0c, x1c)
    dd = (xrep - pxb)
    dd = dd * dd
    li = jax.lax.broadcasted_iota(jnp.int32, (256, 128), 0)
    ki = jax.lax.broadcasted_iota(jnp.int32, (256, 128), 1)
    psum = jnp.where(li // 2 == ki, 1.0, 0.0).astype(jnp.float32)
    r2 = jnp.dot(dd, psum, preferred_element_type=jnp.float32)
    td = tr_ref[:, :] - pt_ref[:, :]
    expo = r2 * inv2s2 - omega * td
    exc = jnp.where(td > 0.0, jnp.exp(expo), 0.0)
    excl = exc.sum(axis=1, keepdims=True)
    Bn = excl.shape[0] // ROWS_PER_EVENT
    exc_n = excl.reshape(Bn, ROWS_PER_EVENT).sum(axis=1, keepdims=True)
    exc_sum = exc_n * (alpha * snorm * omega)

    mu = jnp.dot(cov_ref[:, :], beta_ref[:, :],
                 preferred_element_type=jnp.float32)
    lam = jnp.maximum(mu, EPS) + exc_sum
    log_ref[:, :] = jnp.log(lam + EPS)

    x0 = x_ref[:, 0:1]
    x1 = x_ref[:, 1:2]
    tb = t_ref[:, :]
    g0 = x0 - xg_ref[0:1, :]
    g1 = x1 - xg_ref[1:2, :]
    s_sum = jnp.exp((g0 * g0 + g1 * g1) * inv2s2).sum(axis=1, keepdims=True)
    dtg = tg_ref[0:1, :] - tb
    w = jnp.where(dtg > 0.0, jnp.exp(-omega * dtg), 0.0)
    w_sum = w.sum(axis=1, keepdims=True)
    cross = (s_sum * w_sum).sum(axis=0, keepdims=True)
    cross_ref[0] = cross * (alpha * snorm * omega)

    v = z_ref[:, :] * brep_ref[0:1, :]
    for k in (1, 2, 4, 8):
        v = v + jnp.roll(v, -k, axis=1)
    lane = jax.lax.broadcasted_iota(jnp.int32, v.shape, 1)
    picked = jnp.where(lane % 16 == 0, jnp.maximum(v, EPS), 0.0)
    base_ref[0] = picked.sum(axis=0, keepdims=True).sum(axis=1, keepdims=True)


def kernel(x, t, past_x, past_t, covariates_xt, z_grid, x_grid, t_grid,
           beta, alpha, sigma, omega):
    N, M = past_t.shape
    T, G, D = z_grid.shape
    TG = T * G
    Bn = 128
    NB = N // Bn
    RPE = ROWS_PER_EVENT
    R = Bn * RPE
    Zrows = TG * D // 128
    Zr = Zrows // NB

    px = past_x.reshape(N * RPE, 2 * M // RPE)
    pt2 = past_t.reshape(N * RPE, M // RPE)
    xr = jnp.repeat(x, RPE, axis=0)
    tr = jnp.repeat(t, RPE)[:, None]
    t2 = t[:, None]
    z2 = z_grid.reshape(Zrows, 128)
    xg = x_grid.T
    tg2 = t_grid[None, :]
    beta2 = beta[:, None]
    brep = jnp.tile(beta, 128 // D)[None, :]
    scal = jnp.stack([alpha, sigma, omega]).astype(jnp.float32)[None, :]

    log_int, cross, base = pl.pallas_call(
        _hawkes_body,
        grid=(NB,),
        in_specs=[
            pl.BlockSpec((Bn, 2), lambda i: (i, 0)),
            pl.BlockSpec((Bn, 1), lambda i: (i, 0)),
            pl.BlockSpec((R, 2), lambda i: (i, 0)),
            pl.BlockSpec((R, 1), lambda i: (i, 0)),
            pl.BlockSpec((R, 256), lambda i: (i, 0)),
            pl.BlockSpec((R, 128), lambda i: (i, 0)),
            pl.BlockSpec((Bn, D), lambda i: (i, 0)),
            pl.BlockSpec((Zr, 128), lambda i: (i, 0)),
            pl.BlockSpec((2, G), lambda i: (0, 0)),
            pl.BlockSpec((1, T), lambda i: (0, 0)),
            pl.BlockSpec((D, 1), lambda i: (0, 0)),
            pl.BlockSpec((1, 128), lambda i: (0, 0)),
            pl.BlockSpec((1, 3), lambda i: (0, 0)),
        ],
        out_specs=[
            pl.BlockSpec((Bn, 1), lambda i: (i, 0)),
            pl.BlockSpec((1, 1, 1), lambda i: (i, 0, 0)),
            pl.BlockSpec((1, 1, 1), lambda i: (i, 0, 0)),
        ],
        out_shape=[
            jax.ShapeDtypeStruct((N, 1), jnp.float32),
            jax.ShapeDtypeStruct((NB, 1, 1), jnp.float32),
            jax.ShapeDtypeStruct((NB, 1, 1), jnp.float32),
        ],
        compiler_params=pltpu.CompilerParams(
            dimension_semantics=("parallel",),
        ),
        name="hawkes_fused",
    )(x, t2, xr, tr, px, pt2, covariates_xt, z2, xg, tg2, beta2, brep, scal)

    dxdy = 1.0 / G
    dt_step = t_grid[1] - t_grid[0]
    integral = (base.sum() + cross.sum()) * (dxdy * dt_step)
    return jnp.concatenate([log_int[:, 0], integral[None]])


# speedup vs baseline: 21.9554x; 21.9554x over previous
import jax
import jax.numpy as jnp
from jax.experimental import pallas as pl
from jax.experimental.pallas import tpu as pltpu

TWO_PI = 6.283185307179586
EPS = 1e-6
RPE = 16


def _hawkes_body(x_ref, t_ref, px_ref, pt_ref, cov_ref,
                 z_ref, xg_ref, tg_ref, beta_ref, brep_ref, scal_ref,
                 log_ref, cross_ref, base_ref):
    alpha = scal_ref[0, 0]
    sigma = scal_ref[0, 1]
    omega = scal_ref[0, 2]
    inv2s2 = -0.5 / (sigma * sigma)
    snorm = 1.0 / (TWO_PI * sigma * sigma)

    x0 = x_ref[:, :, 0:1]
    x1 = x_ref[:, :, 1:2]
    tb = t_ref[:, :, :]

    pxb = px_ref[:, :, :]
    Bn = pxb.shape[0]
    parity = jax.lax.broadcasted_iota(jnp.int32, pxb.shape, 2) % 2
    xrep = jnp.where(parity == 0, x0, x1)
    dd = xrep - pxb
    dd = dd * dd
    li = jax.lax.broadcasted_iota(jnp.int32, (256, 128), 0)
    ki = jax.lax.broadcasted_iota(jnp.int32, (256, 128), 1)
    psum = jnp.where(li // 2 == ki, 1.0, 0.0).astype(jnp.float32)
    r2 = jnp.dot(dd.reshape(Bn * RPE, 256), psum,
                 preferred_element_type=jnp.float32)
    td = tb - pt_ref[:, :, :]
    expo = r2.reshape(Bn, RPE, 128) * inv2s2 - omega * td
    exc = jnp.where(td > 0.0, jnp.exp(expo), 0.0)
    excl = exc.sum(axis=2, keepdims=True)
    exc_n = excl.sum(axis=1, keepdims=True)
    exc_sum = exc_n * (alpha * snorm * omega)

    mu = jnp.dot(cov_ref[:, :], beta_ref[:, :],
                 preferred_element_type=jnp.float32)
    lam = jnp.maximum(mu, EPS).reshape(Bn, 1, 1) + exc_sum
    log_ref[:, :, :] = jnp.log(lam + EPS)

    g0 = x0 - xg_ref[:, 0:1, :]
    g1 = x1 - xg_ref[:, 1:2, :]
    s_sum = jnp.exp((g0 * g0 + g1 * g1) * inv2s2).sum(axis=2, keepdims=True)
    dtg = tg_ref[:, :, :] - tb
    w = jnp.where(dtg > 0.0, jnp.exp(-omega * dtg), 0.0)
    w_sum = w.sum(axis=2, keepdims=True)
    cross = (s_sum * w_sum).sum(axis=0, keepdims=True)
    cross_ref[...] = cross * (alpha * snorm * omega)

    v = z_ref[:, :] * brep_ref[0:1, :]
    for k in (1, 2, 4, 8):
        v = v + jnp.roll(v, -k, axis=1)
    lane = jax.lax.broadcasted_iota(jnp.int32, v.shape, 1)
    picked = jnp.where(lane % 16 == 0, jnp.maximum(v, EPS), 0.0)
    base_ref[0] = picked.sum(axis=0, keepdims=True).sum(axis=1, keepdims=True)


def kernel(x, t, past_x, past_t, covariates_xt, z_grid, x_grid, t_grid,
           beta, alpha, sigma, omega):
    N, M = past_t.shape
    T, G, D = z_grid.shape
    TG = T * G
    Bn = 128
    NB = N // Bn
    Zrows = TG * D // 128
    Zr = Zrows // NB

    px = past_x.reshape(N, RPE, 2 * M // RPE)
    pt3 = past_t.reshape(N, RPE, M // RPE)
    x3 = x.reshape(N, 1, 2)
    t3 = t.reshape(N, 1, 1)
    z2 = z_grid.reshape(Zrows, 128)
    xg = x_grid.T.reshape(1, 2, G)
    tg3 = t_grid.reshape(1, 1, T)
    beta2 = beta[:, None]
    brep = jnp.tile(beta, 128 // D)[None, :]
    scal = jnp.stack([alpha, sigma, omega]).astype(jnp.float32)[None, :]

    log_int, cross, base = pl.pallas_call(
        _hawkes_body,
        grid=(NB,),
        in_specs=[
            pl.BlockSpec((Bn, 1, 2), lambda i: (i, 0, 0)),
            pl.BlockSpec((Bn, 1, 1), lambda i: (i, 0, 0)),
            pl.BlockSpec((Bn, RPE, 256), lambda i: (i, 0, 0)),
            pl.BlockSpec((Bn, RPE, 128), lambda i: (i, 0, 0)),
            pl.BlockSpec((Bn, D), lambda i: (i, 0)),
            pl.BlockSpec((Zr, 128), lambda i: (i, 0)),
            pl.BlockSpec((1, 2, G), lambda i: (0, 0, 0)),
            pl.BlockSpec((1, 1, T), lambda i: (0, 0, 0)),
            pl.BlockSpec((D, 1), lambda i: (0, 0)),
            pl.BlockSpec((1, 128), lambda i: (0, 0)),
            pl.BlockSpec((1, 3), lambda i: (0, 0)),
        ],
        out_specs=[
            pl.BlockSpec((Bn, 1, 1), lambda i: (i, 0, 0)),
            pl.BlockSpec((1, 1, 1), lambda i: (i, 0, 0)),
            pl.BlockSpec((1, 1, 1), lambda i: (i, 0, 0)),
        ],
        out_shape=[
            jax.ShapeDtypeStruct((N, 1, 1), jnp.float32),
            jax.ShapeDtypeStruct((NB, 1, 1), jnp.float32),
            jax.ShapeDtypeStruct((NB, 1, 1), jnp.float32),
        ],
        compiler_params=pltpu.CompilerParams(
            dimension_semantics=("parallel",),
        ),
        name="hawkes_fused",
    )(x3, t3, px, pt3, covariates_xt, z2, xg, tg3, beta2, brep, scal)

    dxdy = 1.0 / G
    dt_step = t_grid[1] - t_grid[0]
    integral = (base.sum() + cross.sum()) * (dxdy * dt_step)
    return jnp.concatenate([log_int[:, 0, 0], integral[None]])


# speedup vs baseline: 26.8030x; 1.2208x over previous
import jax
import jax.numpy as jnp
from jax.experimental import pallas as pl
from jax.experimental.pallas import tpu as pltpu

TWO_PI = 6.283185307179586
EPS = 1e-6
CHUNK = 512


def _hawkes_body(x_ref, t_ref, px_ref, pt_ref, cov_ref,
                 z_ref, xg_ref, tg_ref, beta_ref, scal_ref,
                 log_ref, cross_ref, base_ref):
    alpha = scal_ref[0, 0]
    sigma = scal_ref[0, 1]
    omega = scal_ref[0, 2]
    inv2s2 = -0.5 / (sigma * sigma)
    snorm = 1.0 / (TWO_PI * sigma * sigma)

    x0 = x_ref[:, 0:1]
    x1 = x_ref[:, 1:2]
    tb = t_ref[:, :]

    pxb = px_ref[:, :]
    Bn, twoM = pxb.shape
    parity = jax.lax.broadcasted_iota(jnp.int32, pxb.shape, 1) % 2
    xrep = jnp.where(parity == 0, x0, x1)
    dd = xrep - pxb
    dd = dd * dd
    li = jax.lax.broadcasted_iota(jnp.int32, (CHUNK, CHUNK // 2), 0)
    ki = jax.lax.broadcasted_iota(jnp.int32, (CHUNK, CHUNK // 2), 1)
    psum = jnp.where(li // 2 == ki, 1.0, 0.0).astype(jnp.float32)
    acc = jnp.zeros((Bn, CHUNK // 2), jnp.float32)
    for c in range(twoM // CHUNK):
        ddc = dd[:, c * CHUNK:(c + 1) * CHUNK]
        r2c = jnp.dot(ddc, psum, preferred_element_type=jnp.float32)
        tdc = tb - pt_ref[:, c * (CHUNK // 2):(c + 1) * (CHUNK // 2)]
        expoc = r2c * inv2s2 - omega * tdc
        acc = acc + jnp.where(tdc > 0.0, jnp.exp(expoc), 0.0)
    exc_sum = acc.sum(axis=1, keepdims=True) * (alpha * snorm * omega)

    mu = jnp.dot(cov_ref[:, :], beta_ref[:, :],
                 preferred_element_type=jnp.float32)
    lam = jnp.maximum(mu, EPS) + exc_sum
    log_ref[:, :] = jnp.log(lam + EPS)

    g0 = x0 - xg_ref[0:1, :]
    g1 = x1 - xg_ref[1:2, :]
    s_sum = jnp.exp((g0 * g0 + g1 * g1) * inv2s2).sum(axis=1, keepdims=True)
    dtg = tg_ref[0:1, :] - tb
    w = jnp.where(dtg > 0.0, jnp.exp(-omega * dtg), 0.0)
    w_sum = w.sum(axis=1, keepdims=True)
    cross = (s_sum * w_sum).sum(axis=0, keepdims=True)
    cross_ref[0] = cross * (alpha * snorm * omega)

    zb = jnp.dot(z_ref[:, :], beta_ref[:, :],
                 preferred_element_type=jnp.float32)
    base_ref[0] = jnp.maximum(zb, EPS).sum(axis=0, keepdims=True)


def kernel(x, t, past_x, past_t, covariates_xt, z_grid, x_grid, t_grid,
           beta, alpha, sigma, omega):
    N, M = past_t.shape
    T, G, D = z_grid.shape
    TG = T * G
    Bn = 128
    NB = N // Bn
    Zc = TG // NB

    px = past_x.reshape(N, 2 * M)
    t2 = t[:, None]
    z2 = z_grid.reshape(TG, D)
    xg = x_grid.T
    tg2 = t_grid[None, :]
    beta2 = beta[:, None]
    scal = jnp.stack([alpha, sigma, omega]).astype(jnp.float32)[None, :]

    log_int, cross, base = pl.pallas_call(
        _hawkes_body,
        grid=(NB,),
        in_specs=[
            pl.BlockSpec((Bn, 2), lambda i: (i, 0)),
            pl.BlockSpec((Bn, 1), lambda i: (i, 0)),
            pl.BlockSpec((Bn, 2 * M), lambda i: (i, 0)),
            pl.BlockSpec((Bn, M), lambda i: (i, 0)),
            pl.BlockSpec((Bn, D), lambda i: (i, 0)),
            pl.BlockSpec((Zc, D), lambda i: (i, 0)),
            pl.BlockSpec((2, G), lambda i: (0, 0)),
            pl.BlockSpec((1, T), lambda i: (0, 0)),
            pl.BlockSpec((D, 1), lambda i: (0, 0)),
            pl.BlockSpec((1, 3), lambda i: (0, 0)),
        ],
        out_specs=[
            pl.BlockSpec((Bn, 1), lambda i: (i, 0)),
            pl.BlockSpec((1, 1, 1), lambda i: (i, 0, 0)),
            pl.BlockSpec((1, 1, 1), lambda i: (i, 0, 0)),
        ],
        out_shape=[
            jax.ShapeDtypeStruct((N, 1), jnp.float32),
            jax.ShapeDtypeStruct((NB, 1, 1), jnp.float32),
            jax.ShapeDtypeStruct((NB, 1, 1), jnp.float32),
        ],
        compiler_params=pltpu.CompilerParams(
            dimension_semantics=("parallel",),
        ),
        name="hawkes_fused",
    )(x, t2, px, past_t, covariates_xt, z2, xg, tg2, beta2, scal)

    dxdy = 1.0 / G
    dt_step = t_grid[1] - t_grid[0]
    integral = (base.sum() + cross.sum()) * (dxdy * dt_step)
    return jnp.concatenate([log_int[:, 0], integral[None]])


# speedup vs baseline: 33.3643x; 1.2448x over previous
import jax
import jax.numpy as jnp
from jax.experimental import pallas as pl
from jax.experimental.pallas import tpu as pltpu

TWO_PI = 6.283185307179586
EPS = 1e-6


def _hawkes_body(x_ref, t_ref, px0_ref, px1_ref, pt_ref, cov_ref,
                 z_ref, xg_ref, tg_ref, beta_ref, scal_ref,
                 log_ref, cross_ref, base_ref):
    alpha = scal_ref[0, 0]
    sigma = scal_ref[0, 1]
    omega = scal_ref[0, 2]
    inv2s2 = -0.5 / (sigma * sigma)
    snorm = 1.0 / (TWO_PI * sigma * sigma)

    x0 = x_ref[:, 0:1]
    x1 = x_ref[:, 1:2]
    tb = t_ref[:, :]

    d0 = x0 - px0_ref[:, :]
    d1 = x1 - px1_ref[:, :]
    td = tb - pt_ref[:, :]
    expo = (d0 * d0 + d1 * d1) * inv2s2 - omega * td
    exc = jnp.where(td > 0.0, jnp.exp(expo), 0.0)
    exc_sum = exc.sum(axis=1, keepdims=True) * (alpha * snorm * omega)

    mu = jnp.dot(cov_ref[:, :], beta_ref[:, :],
                 preferred_element_type=jnp.float32)
    lam = jnp.maximum(mu, EPS) + exc_sum
    log_ref[:, :] = jnp.log(lam + EPS)

    g0 = x0 - xg_ref[0:1, :]
    g1 = x1 - xg_ref[1:2, :]
    s_sum = jnp.exp((g0 * g0 + g1 * g1) * inv2s2).sum(axis=1, keepdims=True)
    dtg = tg_ref[0:1, :] - tb
    w = jnp.where(dtg > 0.0, jnp.exp(-omega * dtg), 0.0)
    w_sum = w.sum(axis=1, keepdims=True)
    cross = (s_sum * w_sum).sum(axis=0, keepdims=True)
    cross_ref[0] = cross * (alpha * snorm * omega)

    zb = jnp.dot(z_ref[:, :], beta_ref[:, :],
                 preferred_element_type=jnp.float32)
    base_ref[0] = jnp.maximum(zb, EPS).sum(axis=0, keepdims=True)


def kernel(x, t, past_x, past_t, covariates_xt, z_grid, x_grid, t_grid,
           beta, alpha, sigma, omega):
    N, M = past_t.shape
    T, G, D = z_grid.shape
    TG = T * G
    Bn = 128
    NB = N // Bn
    Zc = TG // NB

    pxcat = jnp.concatenate([past_x[:, :, 0], past_x[:, :, 1]], axis=0)
    t2 = t[:, None]
    z2 = z_grid.reshape(TG, D)
    xg = x_grid.T
    tg2 = t_grid[None, :]
    beta2 = beta[:, None]
    scal = jnp.stack([alpha, sigma, omega]).astype(jnp.float32)[None, :]

    log_int, cross, base = pl.pallas_call(
        _hawkes_body,
        grid=(NB,),
        in_specs=[
            pl.BlockSpec((Bn, 2), lambda i: (i, 0)),
            pl.BlockSpec((Bn, 1), lambda i: (i, 0)),
            pl.BlockSpec((Bn, M), lambda i: (i, 0)),
            pl.BlockSpec((Bn, M), lambda i: (i + N // 128, 0)),
            pl.BlockSpec((Bn, M), lambda i: (i, 0)),
            pl.BlockSpec((Bn, D), lambda i: (i, 0)),
            pl.BlockSpec((Zc, D), lambda i: (i, 0)),
            pl.BlockSpec((2, G), lambda i: (0, 0)),
            pl.BlockSpec((1, T), lambda i: (0, 0)),
            pl.BlockSpec((D, 1), lambda i: (0, 0)),
            pl.BlockSpec((1, 3), lambda i: (0, 0)),
        ],
        out_specs=[
            pl.BlockSpec((Bn, 1), lambda i: (i, 0)),
            pl.BlockSpec((1, 1, 1), lambda i: (i, 0, 0)),
            pl.BlockSpec((1, 1, 1), lambda i: (i, 0, 0)),
        ],
        out_shape=[
            jax.ShapeDtypeStruct((N, 1), jnp.float32),
            jax.ShapeDtypeStruct((NB, 1, 1), jnp.float32),
            jax.ShapeDtypeStruct((NB, 1, 1), jnp.float32),
        ],
        compiler_params=pltpu.CompilerParams(
            dimension_semantics=("parallel",),
        ),
        name="hawkes_fused",
    )(x, t2, pxcat, pxcat, past_t, covariates_xt, z2, xg, tg2, beta2, scal)

    dxdy = 1.0 / G
    dt_step = t_grid[1] - t_grid[0]
    integral = (base.sum() + cross.sum()) * (dxdy * dt_step)
    return jnp.concatenate([log_int[:, 0], integral[None]])


# speedup vs baseline: 61.0221x; 1.8290x over previous
import jax
import jax.numpy as jnp
from jax.experimental import pallas as pl
from jax.experimental.pallas import tpu as pltpu

TWO_PI = 6.283185307179586
EPS = 1e-6


def _hawkes_body(x_ref, t_ref, px0_ref, px1_ref, pt_ref, cov_ref,
                 z_ref, bcol_ref, xg_ref, tg_ref, beta_ref, scal_ref,
                 log_ref, cross_ref, base_ref):
    alpha = scal_ref[0, 0]
    sigma = scal_ref[0, 1]
    omega = scal_ref[0, 2]
    inv2s2 = -0.5 / (sigma * sigma)
    snorm = 1.0 / (TWO_PI * sigma * sigma)

    x0 = x_ref[:, 0:1]
    x1 = x_ref[:, 1:2]
    tb = t_ref[:, :]

    d0 = x0 - px0_ref[:, 0, 0, :]
    d1 = x1 - px1_ref[:, 0, 0, :]
    td = tb - pt_ref[:, :]
    expo = (d0 * d0 + d1 * d1) * inv2s2 - omega * td
    exc = jnp.where(td > 0.0, jnp.exp(expo), 0.0)
    exc_sum = exc.sum(axis=1, keepdims=True) * (alpha * snorm * omega)

    mu = jnp.dot(cov_ref[:, :], beta_ref[:, :],
                 preferred_element_type=jnp.float32)
    lam = jnp.maximum(mu, EPS) + exc_sum
    log_ref[:, :] = jnp.log(lam + EPS)

    g0 = x0 - xg_ref[0:1, :]
    g1 = x1 - xg_ref[1:2, :]
    s_sum = jnp.exp((g0 * g0 + g1 * g1) * inv2s2).sum(axis=1, keepdims=True)
    dtg = tg_ref[0:1, :] - tb
    w = jnp.where(dtg > 0.0, jnp.exp(-omega * dtg), 0.0)
    w_sum = w.sum(axis=1, keepdims=True)
    cross = (s_sum * w_sum).sum(axis=0, keepdims=True)
    cross_ref[0] = cross * (alpha * snorm * omega)

    v = z_ref[:, :] * bcol_ref[:, :]
    for k in (1, 2, 4, 8):
        v = v + jnp.roll(v, -k, axis=0)
    row = jax.lax.broadcasted_iota(jnp.int32, v.shape, 0)
    picked = jnp.where(row % 16 == 0, jnp.maximum(v, EPS), 0.0)
    base_ref[0] = picked.sum(axis=1, keepdims=True).sum(axis=0, keepdims=True)


def kernel(x, t, past_x, past_t, covariates_xt, z_grid, x_grid, t_grid,
           beta, alpha, sigma, omega):
    N, M = past_t.shape
    T, G, D = z_grid.shape
    Bn = 128
    NB = N // Bn
    ZR = T * D
    Zr = ZR // NB

    px4 = jnp.transpose(past_x, (0, 2, 1)).reshape(N, 2, 1, M)
    zn = jnp.transpose(z_grid, (0, 2, 1)).reshape(ZR, G)
    t2 = t[:, None]
    xg = x_grid.T
    tg2 = t_grid[None, :]
    beta2 = beta[:, None]
    bcol = jnp.tile(beta, T)[:, None]
    scal = jnp.stack([alpha, sigma, omega]).astype(jnp.float32)[None, :]

    log_int, cross, base = pl.pallas_call(
        _hawkes_body,
        grid=(NB,),
        in_specs=[
            pl.BlockSpec((Bn, 2), lambda i: (i, 0)),
            pl.BlockSpec((Bn, 1), lambda i: (i, 0)),
            pl.BlockSpec((Bn, 1, 1, M), lambda i: (i, 0, 0, 0)),
            pl.BlockSpec((Bn, 1, 1, M), lambda i: (i, 1, 0, 0)),
            pl.BlockSpec((Bn, M), lambda i: (i, 0)),
            pl.BlockSpec((Bn, D), lambda i: (i, 0)),
            pl.BlockSpec((Zr, G), lambda i: (i, 0)),
            pl.BlockSpec((Zr, 1), lambda i: (i, 0)),
            pl.BlockSpec((2, G), lambda i: (0, 0)),
            pl.BlockSpec((1, T), lambda i: (0, 0)),
            pl.BlockSpec((D, 1), lambda i: (0, 0)),
            pl.BlockSpec((1, 3), lambda i: (0, 0)),
        ],
        out_specs=[
            pl.BlockSpec((Bn, 1), lambda i: (i, 0)),
            pl.BlockSpec((1, 1, 1), lambda i: (i, 0, 0)),
            pl.BlockSpec((1, 1, 1), lambda i: (i, 0, 0)),
        ],
        out_shape=[
            jax.ShapeDtypeStruct((N, 1), jnp.float32),
            jax.ShapeDtypeStruct((NB, 1, 1), jnp.float32),
            jax.ShapeDtypeStruct((NB, 1, 1), jnp.float32),
        ],
        compiler_params=pltpu.CompilerParams(
            dimension_semantics=("parallel",),
        ),
        name="hawkes_fused",
    )(x, t2, px4, px4, past_t, covariates_xt, zn, bcol, xg, tg2, beta2, scal)

    dxdy = 1.0 / G
    dt_step = t_grid[1] - t_grid[0]
    integral = (base.sum() + cross.sum()) * (dxdy * dt_step)
    return jnp.concatenate([log_int[:, 0], integral[None]])


# speedup vs baseline: 80.0548x; 1.3119x over previous
import jax
import jax.numpy as jnp
from jax.experimental import pallas as pl
from jax.experimental.pallas import tpu as pltpu

TWO_PI = 6.283185307179586
EPS = 1e-6


def _hawkes_body(x_ref, t_ref, px_ref, pt_ref, cov_ref,
                 z_ref, bcol_ref, xg_ref, tg_ref, beta_ref, scal_ref,
                 log_ref, cross_ref, base_ref):
    alpha = scal_ref[0, 0]
    sigma = scal_ref[0, 1]
    omega = scal_ref[0, 2]
    inv2s2 = -0.5 / (sigma * sigma)
    snorm = 1.0 / (TWO_PI * sigma * sigma)

    x0 = x_ref[:, 0:1]
    x1 = x_ref[:, 1:2]
    tb = t_ref[:, :]

    d0 = x0 - px_ref[:, 0, :]
    d1 = x1 - px_ref[:, 1, :]
    td = tb - pt_ref[:, :]
    expo = (d0 * d0 + d1 * d1) * inv2s2 - omega * td
    exc = jnp.where(td > 0.0, jnp.exp(expo), 0.0)
    exc_sum = exc.sum(axis=1, keepdims=True) * (alpha * snorm * omega)

    mu = jnp.dot(cov_ref[:, :], beta_ref[:, :],
                 preferred_element_type=jnp.float32)
    lam = jnp.maximum(mu, EPS) + exc_sum
    log_ref[:, :] = jnp.log(lam + EPS)

    g0 = x0 - xg_ref[0:1, :]
    g1 = x1 - xg_ref[1:2, :]
    s_sum = jnp.exp((g0 * g0 + g1 * g1) * inv2s2).sum(axis=1, keepdims=True)
    dtg = tg_ref[0:1, :] - tb
    w = jnp.where(dtg > 0.0, jnp.exp(-omega * dtg), 0.0)
    w_sum = w.sum(axis=1, keepdims=True)
    cross = (s_sum * w_sum).sum(axis=0, keepdims=True)
    cross_ref[0] = cross * (alpha * snorm * omega)

    v = z_ref[:, :] * bcol_ref[:, :]
    for k in (1, 2, 4, 8):
        v = v + jnp.roll(v, -k, axis=0)
    row = jax.lax.broadcasted_iota(jnp.int32, v.shape, 0)
    picked = jnp.where(row % 16 == 0, jnp.maximum(v, EPS), 0.0)
    base_ref[0] = picked.sum(axis=1, keepdims=True).sum(axis=0, keepdims=True)


def kernel(x, t, past_x, past_t, covariates_xt, z_grid, x_grid, t_grid,
           beta, alpha, sigma, omega):
    N, M = past_t.shape
    T, G, D = z_grid.shape
    Bn = 128
    NB = N // Bn
    ZR = T * D
    Zr = ZR // NB

    px3 = jnp.transpose(past_x, (0, 2, 1))
    zn = jnp.transpose(z_grid, (0, 2, 1)).reshape(ZR, G)
    t2 = t[:, None]
    xg = x_grid.T
    tg2 = t_grid[None, :]
    beta2 = beta[:, None]
    bcol = jnp.tile(beta, T)[:, None]
    scal = jnp.stack([alpha, sigma, omega]).astype(jnp.float32)[None, :]

    log_int, cross, base = pl.pallas_call(
        _hawkes_body,
        grid=(NB,),
        in_specs=[
            pl.BlockSpec((Bn, 2), lambda i: (i, 0)),
            pl.BlockSpec((Bn, 1), lambda i: (i, 0)),
            pl.BlockSpec((Bn, 2, M), lambda i: (i, 0, 0)),
            pl.BlockSpec((Bn, M), lambda i: (i, 0)),
            pl.BlockSpec((Bn, D), lambda i: (i, 0)),
            pl.BlockSpec((Zr, G), lambda i: (i, 0)),
            pl.BlockSpec((Zr, 1), lambda i: (i, 0)),
            pl.BlockSpec((2, G), lambda i: (0, 0)),
            pl.BlockSpec((1, T), lambda i: (0, 0)),
            pl.BlockSpec((D, 1), lambda i: (0, 0)),
            pl.BlockSpec((1, 3), lambda i: (0, 0)),
        ],
        out_specs=[
            pl.BlockSpec((Bn, 1), lambda i: (i, 0)),
            pl.BlockSpec((1, 1, 1), lambda i: (i, 0, 0)),
            pl.BlockSpec((1, 1, 1), lambda i: (i, 0, 0)),
        ],
        out_shape=[
            jax.ShapeDtypeStruct((N, 1), jnp.float32),
            jax.ShapeDtypeStruct((NB, 1, 1), jnp.float32),
            jax.ShapeDtypeStruct((NB, 1, 1), jnp.float32),
        ],
        compiler_params=pltpu.CompilerParams(
            dimension_semantics=("parallel",),
        ),
        name="hawkes_fused",
    )(x, t2, px3, past_t, covariates_xt, zn, bcol, xg, tg2, beta2, scal)

    dxdy = 1.0 / G
    dt_step = t_grid[1] - t_grid[0]
    integral = (base.sum() + cross.sum()) * (dxdy * dt_step)
    return jnp.concatenate([log_int[:, 0], integral[None]])


# speedup vs baseline: 85.5830x; 1.0691x over previous
import jax
import jax.numpy as jnp
from jax.experimental import pallas as pl
from jax.experimental.pallas import tpu as pltpu

TWO_PI = 6.283185307179586
EPS = 1e-6


def _hawkes_body(x_ref, t_ref, px_ref, pt_ref, cov_ref,
                 z_ref, bcol_ref, xg_ref, tg_ref, beta_ref, scal_ref,
                 log_ref, part_ref):
    alpha = scal_ref[0, 0]
    sigma = scal_ref[0, 1]
    omega = scal_ref[0, 2]
    inv2s2 = -0.5 / (sigma * sigma)
    snorm = 1.0 / (TWO_PI * sigma * sigma)

    x0 = x_ref[:, 0:1]
    x1 = x_ref[:, 1:2]
    tb = t_ref[:, :]

    d0 = x0 - px_ref[:, 0, :]
    d1 = x1 - px_ref[:, 1, :]
    td = tb - pt_ref[:, :]
    expo = (d0 * d0 + d1 * d1) * inv2s2 - omega * td
    exc = jnp.where(td > 0.0, jnp.exp(expo), 0.0)
    exc_sum = exc.sum(axis=1, keepdims=True) * (alpha * snorm * omega)

    mu = jnp.dot(cov_ref[:, :], beta_ref[:, :],
                 preferred_element_type=jnp.float32)
    lam = jnp.maximum(mu, EPS) + exc_sum
    log_ref[:, :] = jnp.log(lam + EPS)

    g0 = x0 - xg_ref[0:1, :]
    g1 = x1 - xg_ref[1:2, :]
    s_sum = jnp.exp((g0 * g0 + g1 * g1) * inv2s2).sum(axis=1, keepdims=True)
    dtg = tg_ref[0:1, :] - tb
    w = jnp.where(dtg > 0.0, jnp.exp(-omega * dtg), 0.0)
    w_sum = w.sum(axis=1, keepdims=True)
    cross = (s_sum * w_sum).sum(axis=0, keepdims=True)

    v = z_ref[:, :] * bcol_ref[:, :]
    for k in (1, 2, 4, 8):
        v = v + jnp.roll(v, -k, axis=0)
    row = jax.lax.broadcasted_iota(jnp.int32, v.shape, 0)
    picked = jnp.where(row % 16 == 0, jnp.maximum(v, EPS), 0.0)
    base = picked.sum(axis=1, keepdims=True).sum(axis=0, keepdims=True)
    part_ref[0] = base + cross * (alpha * snorm * omega)


def kernel(x, t, past_x, past_t, covariates_xt, z_grid, x_grid, t_grid,
           beta, alpha, sigma, omega):
    N, M = past_t.shape
    T, G, D = z_grid.shape
    Bn = 128
    NB = N // Bn
    ZR = T * D
    Zr = ZR // NB

    px3 = jnp.transpose(past_x, (0, 2, 1))
    zn = jnp.transpose(z_grid, (0, 2, 1)).reshape(ZR, G)
    t2 = t[:, None]
    xg = x_grid.T
    tg2 = t_grid[None, :]
    beta2 = beta[:, None]
    bcol = jnp.tile(beta, T)[:, None]
    scal = jnp.stack([alpha, sigma, omega]).astype(jnp.float32)[None, :]

    log_int, part = pl.pallas_call(
        _hawkes_body,
        grid=(NB,),
        in_specs=[
            pl.BlockSpec((Bn, 2), lambda i: (i, 0)),
            pl.BlockSpec((Bn, 1), lambda i: (i, 0)),
            pl.BlockSpec((Bn, 2, M), lambda i: (i, 0, 0)),
            pl.BlockSpec((Bn, M), lambda i: (i, 0)),
            pl.BlockSpec((Bn, D), lambda i: (i, 0)),
            pl.BlockSpec((Zr, G), lambda i: (i, 0)),
            pl.BlockSpec((Zr, 1), lambda i: (i, 0)),
            pl.BlockSpec((2, G), lambda i: (0, 0)),
            pl.BlockSpec((1, T), lambda i: (0, 0)),
            pl.BlockSpec((D, 1), lambda i: (0, 0)),
            pl.BlockSpec((1, 3), lambda i: (0, 0)),
        ],
        out_specs=[
            pl.BlockSpec((Bn, 1), lambda i: (i, 0)),
            pl.BlockSpec((1, 1, 1), lambda i: (i, 0, 0)),
        ],
        out_shape=[
            jax.ShapeDtypeStruct((N, 1), jnp.float32),
            jax.ShapeDtypeStruct((NB, 1, 1), jnp.float32),
        ],
        compiler_params=pltpu.CompilerParams(
            dimension_semantics=("parallel",),
        ),
        name="hawkes_fused",
    )(x, t2, px3, past_t, covariates_xt, zn, bcol, xg, tg2, beta2, scal)

    dxdy = 1.0 / G
    dt_step = t_grid[1] - t_grid[0]
    integral = part.sum() * (dxdy * dt_step)
    return jnp.concatenate([log_int[:, 0], integral[None]])
